# baseline (speedup 1.0000x reference)
import jax
import jax.numpy as jnp
from jax import lax
from jax.experimental import pallas as pl
from jax.experimental.pallas import tpu as pltpu
from jax.experimental.pallas import tpu_sc as plsc

VOCAB = 1000000
EMBED = 64
B = 4096
L = 200

NUM_IDX = B * L
NC, NS = 2, 16
NW = NC * NS
PER_W = NUM_IDX // NW
WIN = 128
NWIN = PER_W // WIN
NBUF = 2
LANES = 2 * EMBED
VREG = 16

assert NWIN % NBUF == 0


TCHUNK = 512


def _tc_pad_transpose(tp):
    def body(in_ref, out_ref):
        out_ref[:, pl.ds(0, EMBED)] = in_ref[...].T

    return pl.pallas_call(
        body,
        grid=(VOCAB // TCHUNK,),
        in_specs=[pl.BlockSpec((EMBED, TCHUNK), lambda i: (0, i))],
        out_specs=pl.BlockSpec((TCHUNK, LANES), lambda i: (i, 0)),
        out_shape=jax.ShapeDtypeStruct((VOCAB, LANES), jnp.float32),
    )(tp)


def _sc_gather(table128, idx):
    mesh = plsc.VectorSubcoreMesh(core_axis_name="c", subcore_axis_name="s")

    scratch = [pltpu.VMEM((PER_W,), jnp.int32)]
    scratch += [pltpu.VMEM((WIN, LANES), jnp.float32) for _ in range(NBUF)]
    scratch += [pltpu.VMEM((WIN, EMBED), jnp.float32) for _ in range(NBUF)]
    scratch += [pltpu.SemaphoreType.DMA for _ in range(2 * NBUF)]

    @pl.kernel(
        out_type=jax.ShapeDtypeStruct((NUM_IDX, EMBED), jnp.float32),
        mesh=mesh,
        scratch_types=scratch,
    )
    def k(table_hbm, idx_hbm, out_hbm, idx_v, *rest):
        wbuf = rest[:NBUF]
        obuf = rest[NBUF:2 * NBUF]
        gsem = rest[2 * NBUF:3 * NBUF]
        ssem = rest[3 * NBUF:4 * NBUF]

        wid = lax.axis_index("s") * NC + lax.axis_index("c")
        base = wid * PER_W
        pltpu.sync_copy(idx_hbm.at[pl.ds(base, PER_W)], idx_v)

        def gfire(j, w):
            pltpu.async_copy(
                table_hbm.at[idx_v.at[pl.ds(w * WIN, WIN)]], wbuf[j], gsem[j]
            )

        def gwait(j, w):
            pltpu.make_async_copy(
                table_hbm.at[idx_v.at[pl.ds(w * WIN, WIN)]], wbuf[j], gsem[j]
            ).wait()

        def sfire(j, w):
            pltpu.async_copy(
                obuf[j], out_hbm.at[pl.ds(base + w * WIN, WIN)], ssem[j]
            )

        def swait(j, w):
            pltpu.make_async_copy(
                obuf[j], out_hbm.at[pl.ds(base + w * WIN, WIN)], ssem[j]
            ).wait()

        def compact(j):
            @pl.loop(0, WIN)
            def _(r):
                for c in range(0, EMBED, VREG):
                    obuf[j][r, pl.ds(c, VREG)] = wbuf[j][r, pl.ds(c, VREG)]

        for j in range(NBUF):
            gfire(j, j)
        for j in range(NBUF):
            gwait(j, j)
            compact(j)
            sfire(j, j)
            gfire(j, NBUF + j)

        @pl.loop(NBUF, NWIN - NBUF, step=NBUF)
        def _(g):
            for j in range(NBUF):
                w = g + j
                swait(j, w - NBUF)
                gwait(j, w)
                compact(j)
                sfire(j, w)
                gfire(j, w + NBUF)

        for j in range(NBUF):
            w = NWIN - NBUF + j
            swait(j, w - NBUF)
            gwait(j, w)
            compact(j)
            sfire(j, w)
        for j in range(NBUF):
            swait(j, NWIN - NBUF + j)

    return k(table128, idx)


def kernel(cat_seq, embed_cat):
    idx = cat_seq.reshape(NUM_IDX).astype(jnp.int32)
    table128 = _tc_pad_transpose(embed_cat.T)
    out = _sc_gather(table128, idx)
    return out.reshape(B, L, EMBED)


# speedup vs baseline: 1.6671x; 1.6671x over previous
import jax
import jax.numpy as jnp
from jax import lax
from jax.experimental import pallas as pl
from jax.experimental.pallas import tpu as pltpu
from jax.experimental.pallas import tpu_sc as plsc

VOCAB = 1000000
EMBED = 64
B = 4096
L = 200

NUM_IDX = B * L
NC, NS = 2, 16
NW = NC * NS
PER_W = NUM_IDX // NW
WIN = 128
NWIN = PER_W // WIN
NBUF = 2
LANES = 2 * EMBED
VREG = 16

assert NWIN % NBUF == 0


TCHUNK = 512


def _tc_pad_transpose(tp):
    def body(in_ref, out_ref):
        out_ref[:, pl.ds(0, EMBED)] = in_ref[...].T

    return pl.pallas_call(
        body,
        grid=(VOCAB // TCHUNK,),
        in_specs=[pl.BlockSpec((EMBED, TCHUNK), lambda i: (0, i))],
        out_specs=pl.BlockSpec((TCHUNK, LANES), lambda i: (i, 0)),
        out_shape=jax.ShapeDtypeStruct((VOCAB, LANES), jnp.float32),
    )(tp)


def _sc_gather(table128, idx):
    mesh = plsc.VectorSubcoreMesh(core_axis_name="c", subcore_axis_name="s")

    scratch = [pltpu.VMEM((PER_W,), jnp.int32)]
    scratch += [pltpu.VMEM((WIN, LANES), jnp.float32) for _ in range(NBUF)]
    scratch += [pltpu.VMEM((WIN, EMBED), jnp.float32) for _ in range(NBUF)]
    scratch += [pltpu.SemaphoreType.DMA for _ in range(2 * NBUF)]

    @pl.kernel(
        out_type=jax.ShapeDtypeStruct((NUM_IDX, EMBED), jnp.float32),
        mesh=mesh,
        scratch_types=scratch,
    )
    def k(table_hbm, idx_hbm, out_hbm, idx_v, *rest):
        wbuf = rest[:NBUF]
        obuf = rest[NBUF:2 * NBUF]
        gsem = rest[2 * NBUF:3 * NBUF]
        ssem = rest[3 * NBUF:4 * NBUF]

        wid = lax.axis_index("s") * NC + lax.axis_index("c")
        base = wid * PER_W
        pltpu.sync_copy(idx_hbm.at[pl.ds(base, PER_W)], idx_v)

        def gfire(j, w):
            pltpu.async_copy(
                table_hbm.at[idx_v.at[pl.ds(w * WIN, WIN)]], wbuf[j], gsem[j]
            )

        def gwait(j, w):
            pltpu.make_async_copy(
                table_hbm.at[idx_v.at[pl.ds(w * WIN, WIN)]], wbuf[j], gsem[j]
            ).wait()

        def sfire(j, w):
            pltpu.async_copy(
                obuf[j], out_hbm.at[pl.ds(base + w * WIN, WIN)], ssem[j]
            )

        def swait(j, w):
            pltpu.make_async_copy(
                obuf[j], out_hbm.at[pl.ds(base + w * WIN, WIN)], ssem[j]
            ).wait()

        def compact(j):
            @pl.loop(0, WIN)
            def _(r):
                for c in range(0, EMBED, VREG):
                    obuf[j][r, pl.ds(c, VREG)] = wbuf[j][r, pl.ds(c, VREG)]

        for j in range(NBUF):
            gfire(j, j)
        for j in range(NBUF):
            gwait(j, j)
            compact(j)
            sfire(j, j)
            gfire(j, NBUF + j)

        @pl.loop(NBUF, NWIN - NBUF, step=NBUF)
        def _(g):
            for j in range(NBUF):
                w = g + j
                swait(j, w - NBUF)
                gwait(j, w)
                compact(j)
                sfire(j, w)
                gfire(j, w + NBUF)

        for j in range(NBUF):
            w = NWIN - NBUF + j
            swait(j, w - NBUF)
            gwait(j, w)
            compact(j)
            sfire(j, w)
        for j in range(NBUF):
            swait(j, NWIN - NBUF + j)

    return k(table128, idx)


def kernel(cat_seq, embed_cat):
    idx = cat_seq.reshape(NUM_IDX).astype(jnp.int32)
    table128 = jnp.pad(embed_cat.T, ((0, LANES - EMBED), (0, 0))).T
    out = _sc_gather(table128, idx)
    return out.reshape(B, L, EMBED)
